# baseline (speedup 1.0000x reference)
import functools

import jax
import jax.numpy as jnp
from jax import lax
from jax.experimental import pallas as pl
from jax.experimental.pallas import tpu as pltpu
from jax.experimental.pallas import tpu_sc as plsc

_NC = 2
_NS = 16
_L = 16
_NW = _NC * _NS

_LOG1P_COEFFS = (
    -0.017208061121084715,
    0.08172680837495,
    -0.18878267362071732,
    0.31459053537083104,
    -0.49697791116761014,
    0.999792435728606,
    3.5075520536942406e-06,
)


def _log1p_poly(u):
    acc = jnp.float32(_LOG1P_COEFFS[0]) * u
    for c in _LOG1P_COEFFS[1:-1]:
        acc = (acc + jnp.float32(c)) * u
    return acc + jnp.float32(_LOG1P_COEFFS[-1])


def _scores_body(d, z_ref, w_ref, b_ref, s1_ref, s2_ref):
    zb = z_ref[...]
    s1_ref[...] = jnp.sum(zb * w_ref[:, :d], axis=1) + b_ref[0]
    s2_ref[...] = jnp.sum(zb * w_ref[:, d:], axis=1)


def _node_scores(z, W, b):
    n, d = z.shape
    blk = 2048
    grid = (n + blk - 1) // blk
    s1, s2 = pl.pallas_call(
        functools.partial(_scores_body, d),
        grid=(grid,),
        in_specs=[
            pl.BlockSpec((blk, d), lambda i: (i, 0)),
            pl.BlockSpec((1, 2 * d), lambda i: (0, 0)),
            pl.BlockSpec(memory_space=pltpu.SMEM),
        ],
        out_specs=[
            pl.BlockSpec((blk,), lambda i: (i,)),
            pl.BlockSpec((blk,), lambda i: (i,)),
        ],
        out_shape=[
            jax.ShapeDtypeStruct((n,), jnp.float32),
            jax.ShapeDtypeStruct((n,), jnp.float32),
        ],
    )(z, W, b)
    return s1, s2


def _weighted_softplus(s1_v, s2_v, i0, i1, j0, j1, wa, wb):
    x = plsc.load_gather(s1_v, [i0]) + plsc.load_gather(s2_v, [i1])
    y = plsc.load_gather(s1_v, [j0]) + plsc.load_gather(s2_v, [j1])
    spx = jnp.maximum(-x, 0.0) + _log1p_poly(jnp.exp(-jnp.abs(x)))
    spy = jnp.maximum(y, 0.0) + _log1p_poly(jnp.exp(-jnp.abs(y)))
    return wa * spx, wb * spy


_TILE = 128


def _edge_loss_body(ch, e, xb0, s1_hbm, s2_hbm, pos_hbm, neg_hbm, w1_hbm,
                    w2_hbm, out_hbm, s1_v, s2_v, pos_v, neg_v,
                    w1_v, w2_v, acc_v, sem):
    wid = lax.axis_index("s") * _NC + lax.axis_index("c")
    base = wid * ch
    copies = [
        pltpu.async_copy(s1_hbm, s1_v.at[pl.ds(0, s1_hbm.shape[0])], sem),
        pltpu.async_copy(s2_hbm, s2_v.at[pl.ds(0, s2_hbm.shape[0])], sem),
        pltpu.async_copy(pos_hbm.at[:, pl.ds(base, ch)],
                         pos_v.at[:, pl.ds(0, ch)], sem),
        pltpu.async_copy(neg_hbm.at[:, pl.ds(base, ch)],
                         neg_v.at[:, pl.ds(0, ch)], sem),
        pltpu.async_copy(w1_hbm.at[pl.ds(base, ch)],
                         w1_v.at[pl.ds(0, ch)], sem),
        pltpu.async_copy(w2_hbm.at[pl.ds(base, ch)],
                         w2_v.at[pl.ds(0, ch)], sem),
    ]
    @pl.when(wid < 2)
    def _():
        xb = xb0 + wid * _TILE
        pltpu.sync_copy(pos_hbm.at[:, pl.ds(xb, _TILE)],
                        pos_v.at[:, pl.ds(ch, _TILE)])
        pltpu.sync_copy(neg_hbm.at[:, pl.ds(xb, _TILE)],
                        neg_v.at[:, pl.ds(ch, _TILE)])
        pltpu.sync_copy(w1_hbm.at[pl.ds(xb, _TILE)],
                        w1_v.at[pl.ds(ch, _TILE)])
        pltpu.sync_copy(w2_hbm.at[pl.ds(xb, _TILE)],
                        w2_v.at[pl.ds(ch, _TILE)])

    for c in copies:
        c.wait()

    def terms(off):
        return _weighted_softplus(
            s1_v, s2_v,
            pos_v[0, pl.ds(off, _L)], pos_v[1, pl.ds(off, _L)],
            neg_v[0, pl.ds(off, _L)], neg_v[1, pl.ds(off, _L)],
            w1_v[pl.ds(off, _L)], w2_v[pl.ds(off, _L)])

    z16 = jnp.zeros((_L,), jnp.float32)

    def _accs_body(i, accs):
        ax0, ay0, ax1, ay1 = accs
        tx0, ty0 = terms(2 * i * _L)
        tx1, ty1 = terms((2 * i + 1) * _L)
        return ax0 + tx0, ay0 + ty0, ax1 + tx1, ay1 + ty1

    accs = plsc.parallel_loop(
        0, ch // (2 * _L), unroll=3, carry=(z16, z16, z16, z16))(_accs_body)

    nxp = jnp.where(wid < 2, (ch + _TILE) // (2 * _L), ch // (2 * _L))
    ax0, ay0, ax1, ay1 = plsc.parallel_loop(
        ch // (2 * _L), nxp, carry=accs)(_accs_body)
    acc = (ax0 + ay0) + (ax1 + ay1)
    acc_v[...] = acc
    pltpu.sync_copy(acc_v, out_hbm.at[wid])


def _edge_loss_partials(s1, s2, pos, neg, w1, w2):
    n_nodes = s1.shape[0]
    e = pos.shape[1]
    n_tiles = e // _TILE
    tiles_per_w = n_tiles // _NW
    ch = tiles_per_w * _TILE
    xb0 = _NW * ch
    assert e % _TILE == 0 and ch % (2 * _L) == 0
    assert e - xb0 <= 2 * _TILE
    mesh = plsc.VectorSubcoreMesh(core_axis_name="c", subcore_axis_name="s")
    fn = functools.partial(
        pl.kernel,
        mesh=mesh,
        compiler_params=pltpu.CompilerParams(needs_layout_passes=False),
        out_type=jax.ShapeDtypeStruct((_NW, _L), jnp.float32),
        scratch_types=[
            pltpu.VMEM((n_nodes,), jnp.float32),
            pltpu.VMEM((n_nodes,), jnp.float32),
            pltpu.VMEM((2, ch + _TILE), jnp.int32),
            pltpu.VMEM((2, ch + _TILE), jnp.int32),
            pltpu.VMEM((ch + _TILE,), jnp.float32),
            pltpu.VMEM((ch + _TILE,), jnp.float32),
            pltpu.VMEM((_L,), jnp.float32),
            pltpu.SemaphoreType.DMA,
        ],
    )(functools.partial(_edge_loss_body, ch, e, xb0))
    return fn(s1, s2, pos, neg, w1, w2)


def kernel(z, pos_edge_index, neg_edge_index, edge_w1, edge_w2, W, b):
    n, d = z.shape
    e = pos_edge_index.shape[1]

    s1, s2 = _node_scores(z, W, b)

    pos = pos_edge_index.astype(jnp.int32)
    neg = neg_edge_index.astype(jnp.int32)
    w1 = lax.squeeze(edge_w1, [1])
    w2 = lax.squeeze(edge_w2, [1])

    partials = _edge_loss_partials(s1, s2, pos, neg, w1, w2)
    return jnp.sum(partials) / jnp.float32(e)


# speedup vs baseline: 1.0053x; 1.0053x over previous
import functools

import jax
import jax.numpy as jnp
from jax import lax
from jax.experimental import pallas as pl
from jax.experimental.pallas import tpu as pltpu
from jax.experimental.pallas import tpu_sc as plsc

_NC = 2
_NS = 16
_L = 16
_NW = _NC * _NS

_LOG1P_COEFFS = (
    -0.017208061121084715,
    0.08172680837495,
    -0.18878267362071732,
    0.31459053537083104,
    -0.49697791116761014,
    0.999792435728606,
    3.5075520536942406e-06,
)


def _log1p_poly(u):
    acc = jnp.float32(_LOG1P_COEFFS[0]) * u
    for c in _LOG1P_COEFFS[1:-1]:
        acc = (acc + jnp.float32(c)) * u
    return acc + jnp.float32(_LOG1P_COEFFS[-1])


def _scores_body(d, z_ref, w_ref, b_ref, s1_ref, s2_ref):
    zb = z_ref[...]
    s1_ref[...] = jnp.sum(zb * w_ref[:, :d], axis=1) + b_ref[0]
    s2_ref[...] = jnp.sum(zb * w_ref[:, d:], axis=1)


def _node_scores(z, W, b):
    n, d = z.shape
    blk = 2048
    grid = (n + blk - 1) // blk
    s1, s2 = pl.pallas_call(
        functools.partial(_scores_body, d),
        grid=(grid,),
        in_specs=[
            pl.BlockSpec((blk, d), lambda i: (i, 0)),
            pl.BlockSpec((1, 2 * d), lambda i: (0, 0)),
            pl.BlockSpec(memory_space=pltpu.SMEM),
        ],
        out_specs=[
            pl.BlockSpec((blk,), lambda i: (i,)),
            pl.BlockSpec((blk,), lambda i: (i,)),
        ],
        out_shape=[
            jax.ShapeDtypeStruct((n,), jnp.float32),
            jax.ShapeDtypeStruct((n,), jnp.float32),
        ],
    )(z, W, b)
    return s1, s2


def _weighted_softplus(s1_v, s2_v, i0, i1, j0, j1, wa, wb):
    x = plsc.load_gather(s1_v, [i0]) + plsc.load_gather(s2_v, [i1])
    y = plsc.load_gather(s1_v, [j0]) + plsc.load_gather(s2_v, [j1])
    spx = jnp.maximum(-x, 0.0) + _log1p_poly(jnp.exp(-jnp.abs(x)))
    spy = jnp.maximum(y, 0.0) + _log1p_poly(jnp.exp(-jnp.abs(y)))
    return wa * spx, wb * spy


_TILE = 128


def _edge_loss_body(ch, e, xb0, s1_hbm, s2_hbm, pos_hbm, neg_hbm, w1_hbm,
                    w2_hbm, out_hbm, s1_v, s2_v, pos_v, neg_v,
                    w1_v, w2_v, acc_v, sem):
    wid = lax.axis_index("s") * _NC + lax.axis_index("c")
    base = wid * ch
    copies = [
        pltpu.async_copy(s1_hbm, s1_v.at[pl.ds(0, s1_hbm.shape[0])], sem),
        pltpu.async_copy(s2_hbm, s2_v.at[pl.ds(0, s2_hbm.shape[0])], sem),
        pltpu.async_copy(pos_hbm.at[:, pl.ds(base, ch)],
                         pos_v.at[:, pl.ds(0, ch)], sem),
        pltpu.async_copy(neg_hbm.at[:, pl.ds(base, ch)],
                         neg_v.at[:, pl.ds(0, ch)], sem),
        pltpu.async_copy(w1_hbm.at[pl.ds(base, ch)],
                         w1_v.at[pl.ds(0, ch)], sem),
        pltpu.async_copy(w2_hbm.at[pl.ds(base, ch)],
                         w2_v.at[pl.ds(0, ch)], sem),
    ]
    @pl.when(wid < 2)
    def _():
        xb = xb0 + wid * _TILE
        pltpu.sync_copy(pos_hbm.at[:, pl.ds(xb, _TILE)],
                        pos_v.at[:, pl.ds(ch, _TILE)])
        pltpu.sync_copy(neg_hbm.at[:, pl.ds(xb, _TILE)],
                        neg_v.at[:, pl.ds(ch, _TILE)])
        pltpu.sync_copy(w1_hbm.at[pl.ds(xb, _TILE)],
                        w1_v.at[pl.ds(ch, _TILE)])
        pltpu.sync_copy(w2_hbm.at[pl.ds(xb, _TILE)],
                        w2_v.at[pl.ds(ch, _TILE)])

    for c in copies:
        c.wait()

    def terms(off):
        return _weighted_softplus(
            s1_v, s2_v,
            pos_v[0, pl.ds(off, _L)], pos_v[1, pl.ds(off, _L)],
            neg_v[0, pl.ds(off, _L)], neg_v[1, pl.ds(off, _L)],
            w1_v[pl.ds(off, _L)], w2_v[pl.ds(off, _L)])

    z16 = jnp.zeros((_L,), jnp.float32)

    def _accs_body(i, accs):
        ax0, ay0, ax1, ay1 = accs
        tx0, ty0 = terms(2 * i * _L)
        tx1, ty1 = terms((2 * i + 1) * _L)
        return ax0 + tx0, ay0 + ty0, ax1 + tx1, ay1 + ty1

    accs = plsc.parallel_loop(
        0, ch // (2 * _L), unroll=2, carry=(z16, z16, z16, z16))(_accs_body)

    nxp = jnp.where(wid < 2, (ch + _TILE) // (2 * _L), ch // (2 * _L))
    ax0, ay0, ax1, ay1 = plsc.parallel_loop(
        ch // (2 * _L), nxp, carry=accs)(_accs_body)
    acc = (ax0 + ay0) + (ax1 + ay1)
    acc_v[...] = acc
    pltpu.sync_copy(acc_v, out_hbm.at[wid])


def _edge_loss_partials(s1, s2, pos, neg, w1, w2):
    n_nodes = s1.shape[0]
    e = pos.shape[1]
    n_tiles = e // _TILE
    tiles_per_w = n_tiles // _NW
    ch = tiles_per_w * _TILE
    xb0 = _NW * ch
    assert e % _TILE == 0 and ch % (2 * _L) == 0
    assert e - xb0 <= 2 * _TILE
    mesh = plsc.VectorSubcoreMesh(core_axis_name="c", subcore_axis_name="s")
    fn = functools.partial(
        pl.kernel,
        mesh=mesh,
        compiler_params=pltpu.CompilerParams(needs_layout_passes=False),
        out_type=jax.ShapeDtypeStruct((_NW, _L), jnp.float32),
        scratch_types=[
            pltpu.VMEM((n_nodes,), jnp.float32),
            pltpu.VMEM((n_nodes,), jnp.float32),
            pltpu.VMEM((2, ch + _TILE), jnp.int32),
            pltpu.VMEM((2, ch + _TILE), jnp.int32),
            pltpu.VMEM((ch + _TILE,), jnp.float32),
            pltpu.VMEM((ch + _TILE,), jnp.float32),
            pltpu.VMEM((_L,), jnp.float32),
            pltpu.SemaphoreType.DMA,
        ],
    )(functools.partial(_edge_loss_body, ch, e, xb0))
    return fn(s1, s2, pos, neg, w1, w2)


def kernel(z, pos_edge_index, neg_edge_index, edge_w1, edge_w2, W, b):
    n, d = z.shape
    e = pos_edge_index.shape[1]

    s1, s2 = _node_scores(z, W, b)

    pos = pos_edge_index.astype(jnp.int32)
    neg = neg_edge_index.astype(jnp.int32)
    w1 = lax.squeeze(edge_w1, [1])
    w2 = lax.squeeze(edge_w2, [1])

    partials = _edge_loss_partials(s1, s2, pos, neg, w1, w2)
    return jnp.sum(partials) / jnp.float32(e)
